# baseline (speedup 1.0000x reference)
import jax
import jax.numpy as jnp
from jax import lax
from jax.experimental import pallas as pl
from jax.experimental.pallas import tpu as pltpu
from jax.experimental.pallas import tpu_sc as plsc

_VOCAB = 100000
_HID = 512
_MAXPOS = 16
_B = 4096
_S = 16

_NC = 2
_NS = 16
_NW = _NC * _NS
_LANES = 16

_TOTAL = _B * _S
_PER_W = _TOTAL // _NW
_CH = 64
_CPW = _PER_W // _CH
_NSLOT = 3


def _emb_body(ids_hbm, table_hbm, pos_hbm, out_hbm,
              idx_v, buf0, buf1, buf2, pos_v,
              gsem0, gsem1, gsem2, osem0, osem1, osem2):
    bufs = (buf0, buf1, buf2)
    gsems = (gsem0, gsem1, gsem2)
    osems = (osem0, osem1, osem2)

    wid = lax.axis_index("s") * _NC + lax.axis_index("c")
    row0 = wid * _PER_W
    chunk0 = wid * _CPW

    pltpu.sync_copy(ids_hbm.at[pl.ds(chunk0, _CPW)], idx_v)
    pltpu.sync_copy(pos_hbm, pos_v)

    def start_gather(c, slot):
        pltpu.async_copy(table_hbm.at[idx_v.at[c]], bufs[slot], gsems[slot])

    def wait_gather(slot):
        pltpu.make_async_copy(table_hbm.at[idx_v.at[0]], bufs[slot],
                              gsems[slot]).wait()

    def start_scatter(c, slot):
        pltpu.async_copy(bufs[slot], out_hbm.at[pl.ds(row0 + c * _CH, _CH)],
                         osems[slot])

    def wait_scatter(slot):
        pltpu.make_async_copy(bufs[slot],
                              out_hbm.at[pl.ds(row0, _CH)], osems[slot]).wait()

    def add_positions(slot):
        buf = bufs[slot]

        @pl.loop(0, _CH)
        def _row(r):
            s = lax.rem(r, _MAXPOS)
            for j in range(_HID // _LANES):
                sl = pl.ds(j * _LANES, _LANES)
                buf[r, sl] = buf[r, sl] + pos_v[s, sl]

    start_gather(0, 0)
    start_gather(1, 1)

    wait_gather(0)
    add_positions(0)
    start_scatter(0, 0)
    start_gather(2, 2)

    @pl.loop(1, 1 + 3 * ((_CPW - 2) // 3), step=3)
    def _section(cc):
        for b in range(3):
            c = cc + b
            slot = (b + 1) % _NSLOT
            wait_gather(slot)
            add_positions(slot)
            start_scatter(c, slot)
            g_slot = (slot + 2) % _NSLOT

            @pl.when(c + 2 < _CPW)
            def _launch():
                wait_scatter(g_slot)
                start_gather(c + 2, g_slot)

    last = _CPW - 1
    slot = last % _NSLOT
    wait_gather(slot)
    add_positions(slot)
    start_scatter(last, slot)
    for c in range(_CPW - 3, _CPW):
        wait_scatter(c % _NSLOT)


def kernel(input_ids, token_table, position_table):
    ids2d = input_ids.reshape(_NW * _CPW, _CH).astype(jnp.int32)
    mesh = plsc.VectorSubcoreMesh(core_axis_name="c", subcore_axis_name="s",
                                  num_cores=_NC, num_subcores=_NS)
    out = pl.kernel(
        _emb_body,
        out_type=jax.ShapeDtypeStruct((_TOTAL, _HID), jnp.float32),
        mesh=mesh,
        scratch_types=[
            pltpu.VMEM((_CPW, _CH), jnp.int32),
            pltpu.VMEM((_CH, _HID), jnp.float32),
            pltpu.VMEM((_CH, _HID), jnp.float32),
            pltpu.VMEM((_CH, _HID), jnp.float32),
            pltpu.VMEM((_MAXPOS, _HID), jnp.float32),
            pltpu.SemaphoreType.DMA,
            pltpu.SemaphoreType.DMA,
            pltpu.SemaphoreType.DMA,
            pltpu.SemaphoreType.DMA,
            pltpu.SemaphoreType.DMA,
            pltpu.SemaphoreType.DMA,
        ],
    )(ids2d, token_table, position_table)
    return out.reshape(_B, _S, _HID)
